# baseline (speedup 1.0000x reference)
import functools

import jax
import jax.numpy as jnp
from jax import lax
from jax.experimental import pallas as pl
from jax.experimental.pallas import tpu as pltpu
from jax.experimental.pallas import tpu_sc as plsc

_NC = 2
_NS = 16
_NW = _NC * _NS
_CH = 128


def _silu(v):
    return v * (1.0 / (1.0 + jnp.exp(-v)))


def _sc_gather(table, idx2):
    two_e = idx2.shape[0]
    per_w = two_e // _NW
    n_chunks = per_w // _CH
    n_pairs = n_chunks // 2
    assert n_chunks == 2 * n_pairs
    rem = per_w - n_chunks * _CH
    mesh = plsc.VectorSubcoreMesh(core_axis_name="c", subcore_axis_name="s")

    @functools.partial(
        pl.kernel,
        out_type=jax.ShapeDtypeStruct((two_e, 128), jnp.float32),
        mesh=mesh,
        scratch_types=[
            pltpu.VMEM((per_w,), jnp.int32),
            pltpu.VMEM((_CH, 128), jnp.float32),
            pltpu.VMEM((_CH, 128), jnp.float32),
            pltpu.VMEM_SHARED((table.shape[0], 128), jnp.float32),
            pltpu.SemaphoreType.DMA,
            pltpu.SemaphoreType.DMA,
            pltpu.SemaphoreType.DMA,
            pltpu.SemaphoreType.DMA,
        ],
    )
    def gather_k(t_hbm, idx_hbm, out_hbm, idx_all, buf0, buf1, tbl, sg0, sg1,
                 sw0, sw1):
        sid = lax.axis_index("s")
        wid = sid * _NC + lax.axis_index("c")
        base0 = wid * per_w
        nrows = table.shape[0]
        trows = (nrows // _NS) & ~7
        tlast = nrows - (_NS - 1) * trows

        @pl.when(sid < _NS - 1)
        def _():
            s = pl.ds(sid * trows, trows)
            pltpu.sync_copy(t_hbm.at[s], tbl.at[s])

        @pl.when(sid == _NS - 1)
        def _():
            s = pl.ds((_NS - 1) * trows, tlast)
            pltpu.sync_copy(t_hbm.at[s], tbl.at[s])

        pltpu.sync_copy(idx_hbm.at[pl.ds(base0, per_w)], idx_all)
        plsc.subcore_barrier()

        def fire(i, buf, sem):
            pltpu.async_copy(tbl.at[idx_all.at[pl.ds(i * _CH, _CH)]], buf, sem)

        def wait(i, buf, sem):
            pltpu.make_async_copy(
                tbl.at[idx_all.at[pl.ds(i * _CH, _CH)]], buf, sem).wait()

        def wb(i, buf, sem):
            pltpu.async_copy(buf, out_hbm.at[pl.ds(base0 + i * _CH, _CH)], sem)

        def wbwait(i, buf, sem):
            pltpu.make_async_copy(
                buf, out_hbm.at[pl.ds(base0 + i * _CH, _CH)], sem).wait()

        fire(0, buf0, sg0)

        def body(o, carry):
            i0 = 2 * o

            @pl.when(o > 0)
            def _():
                wbwait(i0 - 1, buf1, sw1)

            fire(i0 + 1, buf1, sg1)
            wait(i0, buf0, sg0)
            wb(i0, buf0, sw0)
            wait(i0 + 1, buf1, sg1)
            wb(i0 + 1, buf1, sw1)
            wbwait(i0, buf0, sw0)

            @pl.when(o < n_pairs - 1)
            def _():
                fire(i0 + 2, buf0, sg0)

            return carry

        lax.fori_loop(0, n_pairs, body, 0)
        wbwait(2 * n_pairs - 1, buf1, sw1)
        if rem:
            tbase = n_chunks * _CH
            pltpu.sync_copy(
                tbl.at[idx_all.at[pl.ds(tbase, rem)]],
                buf0.at[pl.ds(0, rem)])
            pltpu.sync_copy(buf0.at[pl.ds(0, rem)],
                            out_hbm.at[pl.ds(base0 + tbase, rem)])

    return gather_k(table, idx2)


def _sc_scatter(m2, xc, senders, initm, initx):
    e, hdim = m2.shape
    n = initm.shape[0]
    per_t = e // _NS
    n_chunks = per_t // _CH
    n_pairs = n_chunks // 2
    assert n_chunks == 2 * n_pairs
    rem = per_t - n_chunks * _CH
    mesh = plsc.VectorSubcoreMesh(core_axis_name="c", subcore_axis_name="s")

    @functools.partial(
        pl.kernel,
        out_type=(
            jax.ShapeDtypeStruct((n, hdim), jnp.float32),
            jax.ShapeDtypeStruct((n, hdim), jnp.float32),
        ),
        mesh=mesh,
    scratch_types=[
            pltpu.VMEM((_CH,), jnp.int32),
            pltpu.VMEM((_CH,), jnp.int32),
            pltpu.VMEM((_CH, hdim), jnp.float32),
            pltpu.VMEM((_CH, hdim), jnp.float32),
            pltpu.VMEM((max(rem, 8),), jnp.int32),
            pltpu.VMEM((max(rem, 8), hdim), jnp.float32),
            pltpu.VMEM_SHARED((n, hdim), jnp.float32),
            pltpu.SemaphoreType.DMA,
            pltpu.SemaphoreType.DMA,
        ],
    )
    def scatter_k(m_hbm, x_hbm, idx_hbm, im_hbm, ix_hbm, om_hbm, ox_hbm,
                  idxc0, idxc1, buf0, buf1, idxt, buft, acc, sl0, sl1):
        cid = lax.axis_index("c")
        sid = lax.axis_index("s")
        rows = (n // _NS) & ~7
        last = n - (_NS - 1) * rows

        def sliced(op):
            @pl.when(sid < _NS - 1)
            def _():
                op(pl.ds(sid * rows, rows))

            @pl.when(sid == _NS - 1)
            def _():
                op(pl.ds((_NS - 1) * rows, last))

        @pl.when(cid == 0)
        def _():
            sliced(lambda s: pltpu.sync_copy(im_hbm.at[s], acc.at[s]))

        @pl.when(cid == 1)
        def _():
            sliced(lambda s: pltpu.sync_copy(ix_hbm.at[s], acc.at[s]))

        base0 = sid * per_t
        plsc.subcore_barrier()

        def run(src_hbm):
            def fire(i, buf, idxc, sem):
                pltpu.async_copy(
                    src_hbm.at[pl.ds(base0 + i * _CH, _CH)], buf, sem)
                pltpu.async_copy(
                    idx_hbm.at[pl.ds(base0 + i * _CH, _CH)], idxc, sem)

            def drain(i, buf, idxc, sem):
                pltpu.make_async_copy(
                    src_hbm.at[pl.ds(base0 + i * _CH, _CH)], buf, sem).wait()
                pltpu.make_async_copy(
                    idx_hbm.at[pl.ds(base0 + i * _CH, _CH)], idxc, sem).wait()
                pltpu.sync_copy(buf, acc.at[idxc], add=True)

            fire(0, buf0, idxc0, sl0)

            def body(o, carry):
                i0 = 2 * o
                fire(i0 + 1, buf1, idxc1, sl1)
                drain(i0, buf0, idxc0, sl0)

                @pl.when(o < n_pairs - 1)
                def _():
                    fire(i0 + 2, buf0, idxc0, sl0)

                drain(i0 + 1, buf1, idxc1, sl1)
                return carry

            lax.fori_loop(0, n_pairs, body, 0)
            if rem:
                tbase = base0 + n_chunks * _CH
                pltpu.sync_copy(src_hbm.at[pl.ds(tbase, rem)], buft)
                pltpu.sync_copy(idx_hbm.at[pl.ds(tbase, rem)], idxt)
                pltpu.sync_copy(buft, acc.at[idxt], add=True)

        @pl.when(cid == 0)
        def _():
            run(m_hbm)

        @pl.when(cid == 1)
        def _():
            run(x_hbm)

        plsc.subcore_barrier()

        @pl.when(cid == 0)
        def _():
            sliced(lambda s: pltpu.sync_copy(acc.at[s], om_hbm.at[s]))

        @pl.when(cid == 1)
        def _():
            sliced(lambda s: pltpu.sync_copy(acc.at[s], ox_hbm.at[s]))

    return scatter_k(m2, xc, senders, initm, initx)


def _tc_edge(prows, edge_attr, Ae, Ao, Be, Bo, wd, C4, be1, We2, be2,
             Wx1, bx1, Wx2, bx2, e):
    hdim = 128
    be_blk = 3200
    nb = e // be_blk

    def body(ps_ref, pr_ref, ea_ref, ae_ref, ao_ref, bbe_ref, bbo_ref, wd_ref,
             c4_ref, b1_ref, w2_ref, b2_ref, wx1_ref, bx1_ref, wx2_ref,
             bx2_ref, m_ref, xc_ref):
        ps = ps_ref[...]
        pr = pr_ref[...]
        us = lax.bitcast_convert_type(ps[:, 0:64], jnp.uint32)
        ur = lax.bitcast_convert_type(pr[:, 0:64], jnp.uint32)
        bf = jnp.bfloat16
        hse = lax.bitcast_convert_type(us << 16, jnp.float32).astype(bf)
        hso = lax.bitcast_convert_type(us & jnp.uint32(0xFFFF0000), jnp.float32).astype(bf)
        hre = lax.bitcast_convert_type(ur << 16, jnp.float32).astype(bf)
        hro = lax.bitcast_convert_type(ur & jnp.uint32(0xFFFF0000), jnp.float32).astype(bf)
        dx = ps[:, 64:67] - pr[:, 64:67]
        dist = jnp.sum(dx * dx, axis=1, keepdims=True)
        dot = lambda a, b: jnp.dot(a, b, preferred_element_type=jnp.float32)
        m1 = (dot(hse, ae_ref[...]) + dot(hso, ao_ref[...])
              + dot(hre, bbe_ref[...]) + dot(hro, bbo_ref[...])
              + dist * wd_ref[...]
              + dot(ea_ref[...], c4_ref[...])
              + b1_ref[...])
        m1 = _silu(m1)
        m2 = _silu(dot(m1.astype(bf), w2_ref[...]) + b2_ref[...])
        t = _silu(dot(m2.astype(bf), wx1_ref[...]) + bx1_ref[...])
        w = dot(t.astype(bf), wx2_ref[...]) + bx2_ref[...]
        m_ref[...] = m2
        xc_ref[...] = jnp.concatenate(
            [dx * w, jnp.ones_like(w), jnp.zeros((dx.shape[0], 124), jnp.float32)],
            axis=1)

    full = lambda a: pl.BlockSpec(a.shape, lambda i: (0,) * a.ndim)
    return pl.pallas_call(
        body,
        grid=(nb,),
        in_specs=[
            pl.BlockSpec((be_blk, 128), lambda i: (i, 0)),
            pl.BlockSpec((be_blk, 128), lambda i: (nb + i, 0)),
            pl.BlockSpec((be_blk, 4), lambda i: (i, 0)),
            full(Ae), full(Ao), full(Be), full(Bo), full(wd), full(C4),
            full(be1), full(We2), full(be2), full(Wx1), full(bx1),
            full(Wx2), full(bx2),
        ],
        out_specs=(
            pl.BlockSpec((be_blk, hdim), lambda i: (i, 0)),
            pl.BlockSpec((be_blk, 128), lambda i: (i, 0)),
        ),
        out_shape=(
            jax.ShapeDtypeStruct((e, hdim), jnp.float32),
            jax.ShapeDtypeStruct((e, 128), jnp.float32),
        ),
    )(prows, prows, edge_attr, Ae, Ao, Be, Bo, wd, C4, be1, We2, be2,
      Wx1, bx1, Wx2, bx2)


def _tc_node(h, x, pm, pxc, Wh1, bh1, Wh2, bh2):
    n, hdim = h.shape
    bn = 1000
    nb = n // bn

    def body(h_ref, x_ref, pm_ref, pxc_ref, w1_ref, b1_ref, w2_ref, b2_ref,
             hn_ref, xn_ref):
        hblk = h_ref[...]
        mi = pm_ref[...]
        xc = pxc_ref[...]
        cnt = xc[:, 3:4]
        xn_ref[...] = x_ref[...] + xc[:, 0:3] / cnt
        w1 = w1_ref[...]
        u = _silu(jnp.dot(hblk, w1[0:hdim], preferred_element_type=jnp.float32)
                  + jnp.dot(mi, w1[hdim:], preferred_element_type=jnp.float32)
                  + b1_ref[...])
        hn_ref[...] = hblk + jnp.dot(u, w2_ref[...], preferred_element_type=jnp.float32) \
            + b2_ref[...]

    full = lambda a: pl.BlockSpec(a.shape, lambda i: (0,) * a.ndim)
    return pl.pallas_call(
        body,
        grid=(nb,),
        in_specs=[
            pl.BlockSpec((bn, hdim), lambda i: (i, 0)),
            pl.BlockSpec((bn, 3), lambda i: (i, 0)),
            pl.BlockSpec((bn, hdim), lambda i: (i, 0)),
            pl.BlockSpec((bn, 128), lambda i: (i, 0)),
            full(Wh1), full(bh1), full(Wh2), full(bh2),
        ],
        out_specs=(
            pl.BlockSpec((bn, hdim), lambda i: (i, 0)),
            pl.BlockSpec((bn, 3), lambda i: (i, 0)),
        ),
        out_shape=(
            jax.ShapeDtypeStruct((n, hdim), jnp.float32),
            jax.ShapeDtypeStruct((n, 3), jnp.float32),
        ),
    )(h, x, pm, pxc, Wh1, bh1, Wh2, bh2)


def kernel(edge_index, h, x, edge_attr, We1, be1, We2, be2, Wh1, bh1, Wh2, bh2,
           Wx1, bx1, Wx2, bx2):
    n, hdim = h.shape
    e = edge_index.shape[1]
    e2 = e // 2
    senders = edge_index[0].astype(jnp.int32)
    receivers = edge_index[1].astype(jnp.int32)

    hu = lax.bitcast_convert_type(h.astype(jnp.bfloat16), jnp.uint16)
    packed = hu[:, 0::2].astype(jnp.uint32) | (hu[:, 1::2].astype(jnp.uint32) << 16)
    table = jnp.concatenate(
        [lax.bitcast_convert_type(packed, jnp.float32), x,
         jnp.zeros((n, 61), jnp.float32)], axis=1)

    bf = jnp.bfloat16
    ew = (We1[0:hdim][0::2].astype(bf), We1[0:hdim][1::2].astype(bf),
          We1[hdim:2 * hdim][0::2].astype(bf), We1[hdim:2 * hdim][1::2].astype(bf),
          We1[2 * hdim:2 * hdim + 1], We1[2 * hdim + 1:],
          be1.reshape(1, hdim), We2.astype(bf), be2.reshape(1, hdim),
          Wx1.astype(bf), bx1.reshape(1, hdim), Wx2.astype(bf),
          bx2.reshape(1, 1))

    sA, rA = senders[0:e2], receivers[0:e2]
    sB, rB = senders[e2:], receivers[e2:]
    prowsA = _sc_gather(table, jnp.concatenate([sA, rA]))
    m2A, xcA = _tc_edge(prowsA, edge_attr[0:e2], *ew, e2)
    prowsB = _sc_gather(table, jnp.concatenate([sB, rB]))
    m2B, xcB = _tc_edge(prowsB, edge_attr[e2:], *ew, e2)

    zm = jnp.zeros((n, hdim), jnp.float32)
    pmA, pxA = _sc_scatter(m2A, xcA, sA, zm, zm)
    pmB, pxB = _sc_scatter(m2B, xcB, sB, pmA, pxA)
    return _tc_node(h, x, pmB, pxB,
                    Wh1, bh1.reshape(1, hdim), Wh2, bh2.reshape(1, hdim))


# speedup vs baseline: 1.0208x; 1.0208x over previous
import functools

import jax
import jax.numpy as jnp
from jax import lax
from jax.experimental import pallas as pl
from jax.experimental.pallas import tpu as pltpu
from jax.experimental.pallas import tpu_sc as plsc

_NC = 2
_NS = 16
_NW = _NC * _NS
_CH = 128


def _silu(v):
    return v * (1.0 / (1.0 + jnp.exp(-v)))


def _sc_gather(table, idx2):
    two_e = idx2.shape[0]
    per_w = two_e // _NW
    n_chunks = per_w // _CH
    n_pairs = n_chunks // 2
    assert n_chunks == 2 * n_pairs
    rem = per_w - n_chunks * _CH
    mesh = plsc.VectorSubcoreMesh(core_axis_name="c", subcore_axis_name="s")

    @functools.partial(
        pl.kernel,
        out_type=jax.ShapeDtypeStruct((two_e, 128), jnp.float32),
        mesh=mesh,
        scratch_types=[
            pltpu.VMEM((per_w,), jnp.int32),
            pltpu.VMEM((_CH, 128), jnp.float32),
            pltpu.VMEM((_CH, 128), jnp.float32),
            pltpu.VMEM_SHARED((table.shape[0], 128), jnp.float32),
            pltpu.SemaphoreType.DMA,
            pltpu.SemaphoreType.DMA,
        ],
    )
    def gather_k(t_hbm, idx_hbm, out_hbm, idx_all, buf0, buf1, tbl, sg0, sg1):
        sid = lax.axis_index("s")
        wid = sid * _NC + lax.axis_index("c")
        base0 = wid * per_w
        nrows = table.shape[0]
        trows = (nrows // _NS) & ~7
        tlast = nrows - (_NS - 1) * trows

        @pl.when(sid < _NS - 1)
        def _():
            s = pl.ds(sid * trows, trows)
            pltpu.sync_copy(t_hbm.at[s], tbl.at[s])

        @pl.when(sid == _NS - 1)
        def _():
            s = pl.ds((_NS - 1) * trows, tlast)
            pltpu.sync_copy(t_hbm.at[s], tbl.at[s])

        pltpu.sync_copy(idx_hbm.at[pl.ds(base0, per_w)], idx_all)
        plsc.subcore_barrier()

        def fire(i, buf, sem):
            pltpu.async_copy(tbl.at[idx_all.at[pl.ds(i * _CH, _CH)]], buf, sem)

        def wait(i, buf, sem):
            pltpu.make_async_copy(
                tbl.at[idx_all.at[pl.ds(i * _CH, _CH)]], buf, sem).wait()

        def wb(i, buf):
            pltpu.sync_copy(buf, out_hbm.at[pl.ds(base0 + i * _CH, _CH)])

        fire(0, buf0, sg0)

        def body(o, carry):
            i0 = 2 * o
            fire(i0 + 1, buf1, sg1)
            wait(i0, buf0, sg0)
            wb(i0, buf0)

            @pl.when(o < n_pairs - 1)
            def _():
                fire(i0 + 2, buf0, sg0)

            wait(i0 + 1, buf1, sg1)
            wb(i0 + 1, buf1)
            return carry

        lax.fori_loop(0, n_pairs, body, 0)
        if rem:
            tbase = n_chunks * _CH
            pltpu.sync_copy(
                tbl.at[idx_all.at[pl.ds(tbase, rem)]],
                buf0.at[pl.ds(0, rem)])
            pltpu.sync_copy(buf0.at[pl.ds(0, rem)],
                            out_hbm.at[pl.ds(base0 + tbase, rem)])

    return gather_k(table, idx2)


def _sc_scatter(m2, xc, senders, initm, initx):
    e, hdim = m2.shape
    n = initm.shape[0]
    per_t = e // _NS
    n_chunks = per_t // _CH
    n_pairs = n_chunks // 2
    assert n_chunks == 2 * n_pairs
    rem = per_t - n_chunks * _CH
    mesh = plsc.VectorSubcoreMesh(core_axis_name="c", subcore_axis_name="s")

    @functools.partial(
        pl.kernel,
        out_type=(
            jax.ShapeDtypeStruct((n, hdim), jnp.float32),
            jax.ShapeDtypeStruct((n, hdim), jnp.float32),
        ),
        mesh=mesh,
    scratch_types=[
            pltpu.VMEM((_CH,), jnp.int32),
            pltpu.VMEM((_CH,), jnp.int32),
            pltpu.VMEM((_CH, hdim), jnp.float32),
            pltpu.VMEM((_CH, hdim), jnp.float32),
            pltpu.VMEM((max(rem, 8),), jnp.int32),
            pltpu.VMEM((max(rem, 8), hdim), jnp.float32),
            pltpu.VMEM_SHARED((n, hdim), jnp.float32),
            pltpu.SemaphoreType.DMA,
            pltpu.SemaphoreType.DMA,
        ],
    )
    def scatter_k(m_hbm, x_hbm, idx_hbm, im_hbm, ix_hbm, om_hbm, ox_hbm,
                  idxc0, idxc1, buf0, buf1, idxt, buft, acc, sl0, sl1):
        cid = lax.axis_index("c")
        sid = lax.axis_index("s")
        rows = (n // _NS) & ~7
        last = n - (_NS - 1) * rows

        def sliced(op):
            @pl.when(sid < _NS - 1)
            def _():
                op(pl.ds(sid * rows, rows))

            @pl.when(sid == _NS - 1)
            def _():
                op(pl.ds((_NS - 1) * rows, last))

        @pl.when(cid == 0)
        def _():
            sliced(lambda s: pltpu.sync_copy(im_hbm.at[s], acc.at[s]))

        @pl.when(cid == 1)
        def _():
            sliced(lambda s: pltpu.sync_copy(ix_hbm.at[s], acc.at[s]))

        base0 = sid * per_t
        plsc.subcore_barrier()

        def run(src_hbm):
            def fire(i, buf, idxc, sem):
                pltpu.async_copy(
                    src_hbm.at[pl.ds(base0 + i * _CH, _CH)], buf, sem)
                pltpu.async_copy(
                    idx_hbm.at[pl.ds(base0 + i * _CH, _CH)], idxc, sem)

            def drain(i, buf, idxc, sem):
                pltpu.make_async_copy(
                    src_hbm.at[pl.ds(base0 + i * _CH, _CH)], buf, sem).wait()
                pltpu.make_async_copy(
                    idx_hbm.at[pl.ds(base0 + i * _CH, _CH)], idxc, sem).wait()
                pltpu.sync_copy(buf, acc.at[idxc], add=True)

            fire(0, buf0, idxc0, sl0)

            def body(o, carry):
                i0 = 2 * o
                fire(i0 + 1, buf1, idxc1, sl1)
                drain(i0, buf0, idxc0, sl0)

                @pl.when(o < n_pairs - 1)
                def _():
                    fire(i0 + 2, buf0, idxc0, sl0)

                drain(i0 + 1, buf1, idxc1, sl1)
                return carry

            lax.fori_loop(0, n_pairs, body, 0)
            if rem:
                tbase = base0 + n_chunks * _CH
                pltpu.sync_copy(src_hbm.at[pl.ds(tbase, rem)], buft)
                pltpu.sync_copy(idx_hbm.at[pl.ds(tbase, rem)], idxt)
                pltpu.sync_copy(buft, acc.at[idxt], add=True)

        @pl.when(cid == 0)
        def _():
            run(m_hbm)

        @pl.when(cid == 1)
        def _():
            run(x_hbm)

        plsc.subcore_barrier()

        @pl.when(cid == 0)
        def _():
            sliced(lambda s: pltpu.sync_copy(acc.at[s], om_hbm.at[s]))

        @pl.when(cid == 1)
        def _():
            sliced(lambda s: pltpu.sync_copy(acc.at[s], ox_hbm.at[s]))

    return scatter_k(m2, xc, senders, initm, initx)


def _tc_edge(prows, edge_attr, Ae, Ao, Be, Bo, wd, C4, be1, We2, be2,
             Wx1, bx1, Wx2, bx2, e):
    hdim = 128
    be_blk = 3200
    nb = e // be_blk

    def body(ps_ref, pr_ref, ea_ref, ae_ref, ao_ref, bbe_ref, bbo_ref, wd_ref,
             c4_ref, b1_ref, w2_ref, b2_ref, wx1_ref, bx1_ref, wx2_ref,
             bx2_ref, m_ref, xc_ref):
        ps = ps_ref[...]
        pr = pr_ref[...]
        us = lax.bitcast_convert_type(ps[:, 0:64], jnp.uint32)
        ur = lax.bitcast_convert_type(pr[:, 0:64], jnp.uint32)
        bf = jnp.bfloat16
        hse = lax.bitcast_convert_type(us << 16, jnp.float32).astype(bf)
        hso = lax.bitcast_convert_type(us & jnp.uint32(0xFFFF0000), jnp.float32).astype(bf)
        hre = lax.bitcast_convert_type(ur << 16, jnp.float32).astype(bf)
        hro = lax.bitcast_convert_type(ur & jnp.uint32(0xFFFF0000), jnp.float32).astype(bf)
        dx = ps[:, 64:67] - pr[:, 64:67]
        dist = jnp.sum(dx * dx, axis=1, keepdims=True)
        dot = lambda a, b: jnp.dot(a, b, preferred_element_type=jnp.float32)
        m1 = (dot(hse, ae_ref[...]) + dot(hso, ao_ref[...])
              + dot(hre, bbe_ref[...]) + dot(hro, bbo_ref[...])
              + dist * wd_ref[...]
              + dot(ea_ref[...], c4_ref[...])
              + b1_ref[...])
        m1 = _silu(m1)
        m2 = _silu(dot(m1.astype(bf), w2_ref[...]) + b2_ref[...])
        t = _silu(dot(m2.astype(bf), wx1_ref[...]) + bx1_ref[...])
        w = dot(t.astype(bf), wx2_ref[...]) + bx2_ref[...]
        m_ref[...] = m2
        xc_ref[...] = jnp.concatenate(
            [dx * w, jnp.ones_like(w), jnp.zeros((dx.shape[0], 124), jnp.float32)],
            axis=1)

    full = lambda a: pl.BlockSpec(a.shape, lambda i: (0,) * a.ndim)
    return pl.pallas_call(
        body,
        grid=(nb,),
        in_specs=[
            pl.BlockSpec((be_blk, 128), lambda i: (i, 0)),
            pl.BlockSpec((be_blk, 128), lambda i: (nb + i, 0)),
            pl.BlockSpec((be_blk, 4), lambda i: (i, 0)),
            full(Ae), full(Ao), full(Be), full(Bo), full(wd), full(C4),
            full(be1), full(We2), full(be2), full(Wx1), full(bx1),
            full(Wx2), full(bx2),
        ],
        out_specs=(
            pl.BlockSpec((be_blk, hdim), lambda i: (i, 0)),
            pl.BlockSpec((be_blk, 128), lambda i: (i, 0)),
        ),
        out_shape=(
            jax.ShapeDtypeStruct((e, hdim), jnp.float32),
            jax.ShapeDtypeStruct((e, 128), jnp.float32),
        ),
    )(prows, prows, edge_attr, Ae, Ao, Be, Bo, wd, C4, be1, We2, be2,
      Wx1, bx1, Wx2, bx2)


def _tc_node(h, x, pm, pxc, Wh1, bh1, Wh2, bh2):
    n, hdim = h.shape
    bn = 1000
    nb = n // bn

    def body(h_ref, x_ref, pm_ref, pxc_ref, w1_ref, b1_ref, w2_ref, b2_ref,
             hn_ref, xn_ref):
        hblk = h_ref[...]
        mi = pm_ref[...]
        xc = pxc_ref[...]
        cnt = xc[:, 3:4]
        xn_ref[...] = x_ref[...] + xc[:, 0:3] / cnt
        w1 = w1_ref[...]
        u = _silu(jnp.dot(hblk, w1[0:hdim], preferred_element_type=jnp.float32)
                  + jnp.dot(mi, w1[hdim:], preferred_element_type=jnp.float32)
                  + b1_ref[...])
        hn_ref[...] = hblk + jnp.dot(u, w2_ref[...], preferred_element_type=jnp.float32) \
            + b2_ref[...]

    full = lambda a: pl.BlockSpec(a.shape, lambda i: (0,) * a.ndim)
    return pl.pallas_call(
        body,
        grid=(nb,),
        in_specs=[
            pl.BlockSpec((bn, hdim), lambda i: (i, 0)),
            pl.BlockSpec((bn, 3), lambda i: (i, 0)),
            pl.BlockSpec((bn, hdim), lambda i: (i, 0)),
            pl.BlockSpec((bn, 128), lambda i: (i, 0)),
            full(Wh1), full(bh1), full(Wh2), full(bh2),
        ],
        out_specs=(
            pl.BlockSpec((bn, hdim), lambda i: (i, 0)),
            pl.BlockSpec((bn, 3), lambda i: (i, 0)),
        ),
        out_shape=(
            jax.ShapeDtypeStruct((n, hdim), jnp.float32),
            jax.ShapeDtypeStruct((n, 3), jnp.float32),
        ),
    )(h, x, pm, pxc, Wh1, bh1, Wh2, bh2)


def kernel(edge_index, h, x, edge_attr, We1, be1, We2, be2, Wh1, bh1, Wh2, bh2,
           Wx1, bx1, Wx2, bx2):
    n, hdim = h.shape
    e = edge_index.shape[1]
    e2 = e // 2
    senders = edge_index[0].astype(jnp.int32)
    receivers = edge_index[1].astype(jnp.int32)

    hu = lax.bitcast_convert_type(h.astype(jnp.bfloat16), jnp.uint16)
    packed = hu[:, 0::2].astype(jnp.uint32) | (hu[:, 1::2].astype(jnp.uint32) << 16)
    table = jnp.concatenate(
        [lax.bitcast_convert_type(packed, jnp.float32), x,
         jnp.zeros((n, 61), jnp.float32)], axis=1)

    bf = jnp.bfloat16
    ew = (We1[0:hdim][0::2].astype(bf), We1[0:hdim][1::2].astype(bf),
          We1[hdim:2 * hdim][0::2].astype(bf), We1[hdim:2 * hdim][1::2].astype(bf),
          We1[2 * hdim:2 * hdim + 1], We1[2 * hdim + 1:],
          be1.reshape(1, hdim), We2.astype(bf), be2.reshape(1, hdim),
          Wx1.astype(bf), bx1.reshape(1, hdim), Wx2.astype(bf),
          bx2.reshape(1, 1))

    sA, rA = senders[0:e2], receivers[0:e2]
    sB, rB = senders[e2:], receivers[e2:]
    prowsA = _sc_gather(table, jnp.concatenate([sA, rA]))
    m2A, xcA = _tc_edge(prowsA, edge_attr[0:e2], *ew, e2)
    prowsB = _sc_gather(table, jnp.concatenate([sB, rB]))
    m2B, xcB = _tc_edge(prowsB, edge_attr[e2:], *ew, e2)

    zm = jnp.zeros((n, hdim), jnp.float32)
    pmA, pxA = _sc_scatter(m2A, xcA, sA, zm, zm)
    pmB, pxB = _sc_scatter(m2B, xcB, sB, pmA, pxA)
    return _tc_node(h, x, pmB, pxB,
                    Wh1, bh1.reshape(1, hdim), Wh2, bh2.reshape(1, hdim))
